# baseline (speedup 1.0000x reference)
import functools

import jax
import jax.numpy as jnp
from jax import lax
from jax.experimental import pallas as pl
from jax.experimental.pallas import tpu as pltpu
from jax.experimental.pallas import tpu_sc as plsc

_K = 256
_ROWS = 131072
_C = 81

_NSC = 8192
_W = 256
_NTILES = 32
_CPT = _NSC // _NTILES

_R = 8192
_NTC = _ROWS - _NSC
_GTC = _NTC // _R
_GPAD = 16


def _sc_kernel():
    mesh = plsc.VectorSubcoreMesh(core_axis_name="c", subcore_axis_name="s")

    @functools.partial(
        pl.kernel,
        out_type=jax.ShapeDtypeStruct((_NSC,), jnp.float32),
        mesh=mesh,
        scratch_types=[
            pltpu.VMEM((_C, _W), jnp.float32),
            pltpu.VMEM((_W,), jnp.int32),
            pltpu.VMEM((_W,), jnp.float32),
        ],
        compiler_params=pltpu.CompilerParams(use_tc_tiling_on_sc=True),
    )
    def sc_part(dt_hbm, lab_hbm, out_hbm, xbuf, lbuf, rbuf):
        wid = lax.axis_index("s") * 2 + lax.axis_index("c")
        base = wid * _CPT

        def chunk(ci, _):
            c0 = base + ci * _W
            pltpu.sync_copy(dt_hbm.at[:, pl.ds(c0, _W)], xbuf)
            pltpu.sync_copy(lab_hbm.at[0, pl.ds(c0, _W)], lbuf)

            def grp(g, _):
                off = g * 16
                lab16 = lbuf[pl.ds(off, 16)]
                acc = jnp.zeros((16,), jnp.float32)
                el = jnp.zeros((16,), jnp.float32)
                for j in range(_C):
                    ev = jnp.exp(xbuf[j, pl.ds(off, 16)])
                    acc = acc + ev
                    el = jnp.where(lab16 == j, ev, el)
                rbuf[pl.ds(off, 16)] = acc / el
                return 0

            lax.fori_loop(0, _W // 16, grp, 0)
            pltpu.sync_copy(rbuf, out_hbm.at[pl.ds(c0, _W)])
            return 0

        lax.fori_loop(0, _CPT // _W, chunk, 0)

    return sc_part


def _tc_body(d_ref, l_ref, r_ref, r_sc):
    i = pl.program_id(0)
    x = d_ref[...]
    e = jnp.exp(x)
    onehot = lax.broadcasted_iota(jnp.int32, (_C, _R), 0) == l_ref[...]
    me = jnp.where(onehot, e, 0.0)
    ones = jnp.ones((1, _C), jnp.float32)
    dn = (((1,), (0,)), ((), ()))
    s_row = lax.dot_general(
        ones, e, dn, preferred_element_type=jnp.float32)
    e_row = lax.dot_general(
        ones, me, dn, preferred_element_type=jnp.float32)
    row = pl.multiple_of(i * 8, 8)
    r_sc[pl.ds(row, 1), :] = s_row / e_row

    @pl.when(i == _GTC - 1)
    def _():
        rows = [r_sc[pl.ds(8 * k, 1), :] for k in range(_GTC)]
        rows.append(jnp.ones((_GPAD - _GTC, _R), jnp.float32))
        r_ref[...] = jnp.concatenate(rows, axis=0)


def _merge_body(a_ref, b_ref, out_ref):
    a = jnp.maximum(a_ref[...], 1.0)
    b = jnp.maximum(b_ref[...], 1.0)
    ba = lax.bitcast_convert_type(a, jnp.int32)
    bb = lax.bitcast_convert_type(b, jnp.int32)

    def it(_, lohi):
        lo, hi = lohi
        mid = lo + ((hi - lo) >> 1)
        cnt = (jnp.sum((ba >= mid).astype(jnp.int32))
               + jnp.sum((bb >= mid).astype(jnp.int32)))
        big = cnt >= _K
        return (jnp.where(big, mid, lo), jnp.where(big, hi, mid))

    lo, _ = lax.fori_loop(
        0, 31, it, (jnp.int32(0), jnp.int32(0x7F800001)), unroll=False)

    ea = jnp.log(a)
    eb = jnp.log(b)
    gt_a, gt_b = ba > lo, bb > lo
    eq_a, eq_b = ba == lo, bb == lo
    c_gt = (jnp.sum(gt_a.astype(jnp.int32))
            + jnp.sum(gt_b.astype(jnp.int32))).astype(jnp.float32)
    c_eq = (jnp.sum(eq_a.astype(jnp.int32))
            + jnp.sum(eq_b.astype(jnp.int32))).astype(jnp.float32)
    s_gt = jnp.sum(jnp.where(gt_a, ea, 0.0)) + jnp.sum(jnp.where(gt_b, eb, 0.0))
    s_eq = jnp.sum(jnp.where(eq_a, ea, 0.0)) + jnp.sum(jnp.where(eq_b, eb, 0.0))
    loss = (s_gt + (_K - c_gt) * (s_eq / c_eq)) / _K
    out_ref[0, 0] = loss


def kernel(dists, labels):
    dt = dists.T
    lab = labels.reshape(1, _ROWS).astype(jnp.int32)

    r_sc = _sc_kernel()(dt, lab)

    r_tc = pl.pallas_call(
        _tc_body,
        grid=(_GTC,),
        in_specs=[
            pl.BlockSpec((_C, _R), lambda i: (0, i + _NSC // _R)),
            pl.BlockSpec((1, _R), lambda i: (0, i + _NSC // _R)),
        ],
        out_specs=pl.BlockSpec((_GPAD, _R), lambda i: (0, 0)),
        out_shape=jax.ShapeDtypeStruct((_GPAD, _R), jnp.float32),
        scratch_shapes=[pltpu.VMEM((8 * _GTC, _R), jnp.float32)],
    )(dt, lab)

    loss = pl.pallas_call(
        _merge_body,
        in_specs=[
            pl.BlockSpec(memory_space=pltpu.MemorySpace.VMEM),
            pl.BlockSpec(memory_space=pltpu.MemorySpace.VMEM),
        ],
        out_specs=pl.BlockSpec(memory_space=pltpu.MemorySpace.SMEM),
        out_shape=jax.ShapeDtypeStruct((1, 1), jnp.float32),
    )(r_tc, r_sc.reshape(_NSC // 128, 128))
    return loss[0, 0]


# speedup vs baseline: 1.7532x; 1.7532x over previous
import jax
import jax.numpy as jnp
from jax.experimental import pallas as pl
from jax.experimental.pallas import tpu as pltpu

_K = 256
_ROWS = 131072
_C = 81
_R = 16384
_G = _ROWS // _R


def _body(d_ref, l_ref, out_ref, s_sc, e_sc):
    i = pl.program_id(0)
    x = d_ref[...]
    e = jnp.exp(x)
    onehot = jax.lax.broadcasted_iota(jnp.int32, (_C, _R), 0) == l_ref[...]
    me = jnp.where(onehot, e, 0.0)
    ones = jnp.ones((1, _C), jnp.float32)
    dn = (((1,), (0,)), ((), ()))
    s_row = jax.lax.dot_general(
        ones, e, dn, preferred_element_type=jnp.float32)
    e_row = jax.lax.dot_general(
        ones, me, dn, preferred_element_type=jnp.float32)
    row = pl.multiple_of(i * 8, 8)
    s_sc[pl.ds(row, 1), :] = s_row
    e_sc[pl.ds(row, 1), :] = e_row

    @pl.when(i == _G - 1)
    def _():
        s = jnp.concatenate(
            [s_sc[pl.ds(8 * k, 1), :] for k in range(_G)], axis=0)
        ee = jnp.concatenate(
            [e_sc[pl.ds(8 * k, 1), :] for k in range(_G)], axis=0)
        r = jnp.maximum(s / ee, 1.0)
        bits = jax.lax.bitcast_convert_type(r, jnp.int32)

        def it(_, lohi):
            lo, hi = lohi
            mid = lo + ((hi - lo) >> 1)
            cnt = jnp.sum((bits >= mid).astype(jnp.int32))
            big = cnt >= _K
            return (jnp.where(big, mid, lo), jnp.where(big, hi, mid))

        lo, _ = jax.lax.fori_loop(
            0, 31, it, (jnp.int32(0), jnp.int32(0x7F800001)), unroll=False)

        ent = jnp.log(r)
        gt = bits > lo
        eq = bits == lo
        c_gt = jnp.sum(gt.astype(jnp.int32)).astype(jnp.float32)
        c_eq = jnp.sum(eq.astype(jnp.int32)).astype(jnp.float32)
        s_gt = jnp.sum(jnp.where(gt, ent, 0.0))
        s_eq = jnp.sum(jnp.where(eq, ent, 0.0))
        loss = (s_gt + (_K - c_gt) * (s_eq / c_eq)) / _K
        out_ref[0, 0] = loss


def kernel(dists, labels):
    dt = dists.T
    lab = labels.reshape(1, _ROWS).astype(jnp.int32)
    loss = pl.pallas_call(
        _body,
        grid=(_G,),
        in_specs=[
            pl.BlockSpec((_C, _R), lambda i: (0, i)),
            pl.BlockSpec((1, _R), lambda i: (0, i)),
        ],
        out_specs=pl.BlockSpec(
            (1, 1), lambda i: (0, 0), memory_space=pltpu.MemorySpace.SMEM),
        out_shape=jax.ShapeDtypeStruct((1, 1), jnp.float32),
        scratch_shapes=[
            pltpu.VMEM((8 * _G, _R), jnp.float32),
            pltpu.VMEM((8 * _G, _R), jnp.float32),
        ],
    )(dt, lab)
    return loss[0, 0]
